# Initial kernel scaffold; baseline (speedup 1.0000x reference)
import jax
import jax.numpy as jnp
from jax.experimental import pallas as pl


def kernel(x, edge_index, edge_attr, batch, fc1_W, fc1_b, fc2_W, fc2_b, fc3_W, fc3_b, c1_W, c1_as, c1_ad, c1_b, c2_W, c2_as, c2_ad, c2_b, c3_W, c3_as, c3_ad, c3_b, p1_Wr, p1_br, p1_Wo, p2_Wr, p2_br, p2_Wo, p3_Wr, p3_br, p3_Wo, bn1_g, bn1_b, bn2_g, bn2_b, bn3_g, bn3_b):
    raise NotImplementedError("write your pallas kernel here")


# speedup vs baseline: 4.9279x; 4.9279x over previous
import functools
import math

import jax
import jax.numpy as jnp
from jax import lax
from jax.experimental import pallas as pl
from jax.experimental.pallas import tpu as pltpu
from jax.experimental.pallas import tpu_sc as plsc

F32 = jnp.float32
I32 = jnp.int32

NW = 32
CH = 128
NCH = 80
EPT = NCH * CH
EP = NW * EPT
E_REAL = 320000


def _nr(n):
    return ((n + 127) // 128) * 128


def _nd2(n):
    return ((n + 32 + 15) // 16) * 16


def _sc_mesh():
    return plsc.VectorSubcoreMesh(core_axis_name="c", subcore_axis_name="s")


@functools.lru_cache(maxsize=None)
def _gat_edge_kernel(N, NR, ND):
    NCHT = EP // (16 * CH)
    NDA = max(NR, ND)
    NB = ND // 128
    REM = ND - NB * 128
    NBT = (NB + 15) // 16

    @functools.partial(
        pl.kernel,
        mesh=_sc_mesh(),
        compiler_params=pltpu.CompilerParams(use_tc_tiling_on_sc=False, needs_layout_passes=False),
        out_type=[jax.ShapeDtypeStruct((2, ND, 64), F32),
                  jax.ShapeDtypeStruct((2, ND), F32)],
        scratch_types=[
            pltpu.VMEM((NCHT, CH), I32),
            pltpu.VMEM((NCHT, CH), I32),
            pltpu.VMEM((NCHT * CH,), F32),
            pltpu.VMEM((NDA,), F32),
            pltpu.VMEM((NDA,), F32),
            pltpu.VMEM((16,), F32),
            pltpu.VMEM((CH, 64), F32),
            pltpu.VMEM_SHARED((ND, 64), F32),
            pltpu.VMEM_SHARED((ND,), F32),
            pltpu.SemaphoreType.DMA,
        ],
    )
    def k(s_hbm, t_hbm, as_hbm, ad_hbm, h0_hbm, h1_hbm, b_hbm, out_hbm,
          den_hbm, sv, tv, wv, asv, adv, bv, rows, out_sh, den_sh, sem):
        cid = lax.axis_index("c")
        sid = lax.axis_index("s")

        pltpu.sync_copy(s_hbm.at[sid], sv)
        pltpu.sync_copy(t_hbm.at[sid], tv)
        pltpu.sync_copy(as_hbm, asv.at[pl.ds(0, NR)])
        pltpu.sync_copy(ad_hbm, adv.at[pl.ds(0, NR)])
        pltpu.sync_copy(b_hbm, bv)
        z16 = jnp.zeros((16,), F32)
        for i in range((NDA - NR) // 16):
            asv[pl.ds(NR + i * 16, 16)] = z16
            adv[pl.ds(NR + i * 16, 16)] = z16

        def zrow(i, c):
            rows[i >> 2, pl.ds((i & 3) * 16, 16)] = z16
            return c
        lax.fori_loop(0, CH * 4, zrow, 0)
        def zw(i, c):
            wv[pl.ds(i * 16, 16)] = z16
            return c
        lax.fori_loop(0, NCHT * CH // 16, zw, 0)
        for b in range(NBT):
            blk = b * 16 + sid
            @pl.when(blk < NB)
            def _():
                pltpu.sync_copy(rows, out_sh.at[pl.ds(blk * 128, 128)])
        if REM:
            @pl.when(sid == 15)
            def _():
                pltpu.sync_copy(rows.at[pl.ds(0, REM)],
                                out_sh.at[pl.ds(NB * 128, REM)])
        @pl.when(sid == 0)
        def _():
            pltpu.sync_copy(wv.at[pl.ds(0, ND)], den_sh)
        plsc.subcore_barrier()

        bscal = bv[...][0]

        def mainloop(h_hbm, do_den):
            def chunk(j, c):
                cp = pltpu.async_copy(h_hbm.at[sv.at[j]], rows, sem)

                def wgrp(i, c2):
                    svv = sv[j, pl.ds(i * 16, 16)]
                    tvv = tv[j, pl.ds(i * 16, 16)]
                    a = plsc.load_gather(asv, [svv])
                    b2 = plsc.load_gather(adv, [tvv])
                    e = a + b2
                    e = jnp.where(e > 0, e, e * 0.2)
                    w = jnp.exp(e - bscal)
                    wv[pl.ds(j * CH + i * 16, 16)] = w
                    return c2
                lax.fori_loop(0, CH // 16, wgrp, 0)
                cp.wait()

                def scale(e2, c3):
                    wbc = plsc.load_gather(wv, [jnp.full((16,), j * CH + e2, I32)])
                    def f4(kk, c4):
                        rows[e2, pl.ds(kk * 16, 16)] = rows[e2, pl.ds(kk * 16, 16)] * wbc
                        return c4
                    lax.fori_loop(0, 4, f4, 0)
                    return c3
                lax.fori_loop(0, CH, scale, 0)

                pltpu.sync_copy(rows, out_sh.at[tv.at[j]], add=True)
                if do_den:
                    pltpu.sync_copy(wv.at[pl.ds(j * CH, CH)],
                                    den_sh.at[tv.at[j]], add=True)
                return c
            lax.fori_loop(0, NCHT, chunk, 0)

        @pl.when(cid == 0)
        def _():
            mainloop(h0_hbm, True)
        @pl.when(cid == 1)
        def _():
            mainloop(h1_hbm, False)

        plsc.subcore_barrier()
        for b in range(NBT):
            blk = b * 16 + sid
            @pl.when(blk < NB)
            def _():
                pltpu.sync_copy(out_sh.at[pl.ds(blk * 128, 128)],
                                out_hbm.at[cid, pl.ds(blk * 128, 128)])
        if REM:
            @pl.when(sid == 15)
            def _():
                pltpu.sync_copy(out_sh.at[pl.ds(NB * 128, REM)],
                                out_hbm.at[cid, pl.ds(NB * 128, REM)])
        @pl.when(sid == 0)
        def _():
            pltpu.sync_copy(den_sh, den_hbm.at[cid])
    return k


@functools.lru_cache(maxsize=None)
def _pool_agg_kernel(N, NR, ND):
    @functools.partial(
        pl.kernel,
        mesh=_sc_mesh(),
        compiler_params=pltpu.CompilerParams(use_tc_tiling_on_sc=False, needs_layout_passes=False),
        out_type=[jax.ShapeDtypeStruct((2, ND), F32)],
        scratch_types=[
            pltpu.VMEM((NCH, CH), I32),
            pltpu.VMEM((NCH, CH), I32),
            pltpu.VMEM((NCH * CH,), F32),
            pltpu.VMEM((ND,), F32),
            pltpu.VMEM_SHARED((ND,), F32),
        ],
    )
    def k(s_hbm, t_hbm, r_hbm, agg_hbm, sv, tv, wv, rv, agg_sh):
        cid = lax.axis_index("c")
        sid = lax.axis_index("s")
        wid = sid * 2 + cid
        pltpu.sync_copy(s_hbm.at[wid], sv)
        pltpu.sync_copy(t_hbm.at[wid], tv)
        pltpu.sync_copy(r_hbm, rv.at[pl.ds(0, NR)])
        z16 = jnp.zeros((16,), F32)
        def zw(i, c):
            wv[pl.ds(i * 16, 16)] = z16
            return c
        lax.fori_loop(0, NCH * CH // 16, zw, 0)
        @pl.when(sid == 0)
        def _():
            pltpu.sync_copy(wv.at[pl.ds(0, ND)], agg_sh)
        plsc.subcore_barrier()

        def chunk(j, c):
            def wgrp(i, c2):
                svv = sv[j, pl.ds(i * 16, 16)]
                w = plsc.load_gather(rv, [svv])
                wv[pl.ds(j * CH + i * 16, 16)] = w
                return c2
            lax.fori_loop(0, CH // 16, wgrp, 0)
            pltpu.sync_copy(wv.at[pl.ds(j * CH, CH)], agg_sh.at[tv.at[j]], add=True)
            return c
        lax.fori_loop(0, NCH, chunk, 0)

        plsc.subcore_barrier()
        @pl.when(sid == 0)
        def _():
            pltpu.sync_copy(agg_sh, agg_hbm.at[cid])
    return k


@functools.lru_cache(maxsize=None)
def _remap_kernel(N, NR, ND, KN):
    @functools.partial(
        pl.kernel,
        mesh=_sc_mesh(),
        compiler_params=pltpu.CompilerParams(use_tc_tiling_on_sc=False, needs_layout_passes=False),
        out_type=[jax.ShapeDtypeStruct((NW, NCH, CH), I32),
                  jax.ShapeDtypeStruct((NW, NCH, CH), I32)],
        scratch_types=[
            pltpu.VMEM((NCH, CH), I32),
            pltpu.VMEM((NCH, CH), I32),
            pltpu.VMEM((NCH, CH), I32),
            pltpu.VMEM((NCH, CH), I32),
            pltpu.VMEM((ND,), I32),
        ],
    )
    def k(s_hbm, t_hbm, nix_hbm, so_hbm, to_hbm, sv, tv, sov, tov, nixv):
        cid = lax.axis_index("c")
        sid = lax.axis_index("s")
        wid = sid * 2 + cid
        pltpu.sync_copy(s_hbm.at[wid], sv)
        pltpu.sync_copy(t_hbm.at[wid], tv)
        pltpu.sync_copy(nix_hbm, nixv.at[pl.ds(0, NR)])
        m16 = jnp.full((16,), -1, I32)
        for i in range((ND - NR) // 16):
            nixv[pl.ds(NR + i * 16, 16)] = m16
        lanes = lax.iota(I32, 16)

        def chunk(j, c):
            def grp(i, c2):
                svv = sv[j, pl.ds(i * 16, 16)]
                tvv = tv[j, pl.ds(i * 16, 16)]
                ns = plsc.load_gather(nixv, [svv])
                nt = plsc.load_gather(nixv, [tvv])
                ok = (ns >= 0) & (nt >= 0)
                dummy = KN + ((i * 16 + lanes) & 31)
                sov[j, pl.ds(i * 16, 16)] = jnp.where(ok, ns, 0)
                tov[j, pl.ds(i * 16, 16)] = jnp.where(ok, nt, dummy)
                return c2
            lax.fori_loop(0, CH // 16, grp, 0)
            return c
        lax.fori_loop(0, NCH, chunk, 0)
        pltpu.sync_copy(sov, so_hbm.at[wid])
        pltpu.sync_copy(tov, to_hbm.at[wid])
    return k


@functools.lru_cache(maxsize=None)
def _compact_kernel(NP2, NCHD, KN):
    @functools.partial(
        pl.kernel,
        mesh=_sc_mesh(),
        compiler_params=pltpu.CompilerParams(use_tc_tiling_on_sc=False, needs_layout_passes=False),
        out_type=[jax.ShapeDtypeStruct((KN + 128, 128), F32)],
        scratch_types=[
            pltpu.VMEM((NCHD, CH), I32),
            pltpu.VMEM((NCHD * CH,), F32),
            pltpu.VMEM((CH, 128), F32),
        ],
    )
    def k(x_hbm, g_hbm, scat_hbm, y_hbm, scv, gv, rows):
        cid = lax.axis_index("c")
        sid = lax.axis_index("s")
        wid = sid * 2 + cid
        base = wid * NCHD * CH
        pltpu.sync_copy(scat_hbm.at[wid], scv)
        pltpu.sync_copy(g_hbm.at[pl.ds(base, NCHD * CH)], gv)

        def chunk(j, c):
            pltpu.sync_copy(x_hbm.at[pl.ds(base + j * CH, CH)], rows)
            def scale(e2, c3):
                wbc = plsc.load_gather(gv, [jnp.full((16,), j * CH + e2, I32)])
                def f8(kk, c4):
                    rows[e2, pl.ds(kk * 16, 16)] = rows[e2, pl.ds(kk * 16, 16)] * wbc
                    return c4
                lax.fori_loop(0, 8, f8, 0)
                return c3
            lax.fori_loop(0, CH, scale, 0)
            pltpu.sync_copy(rows, y_hbm.at[scv.at[j]])
            return c
        lax.fori_loop(0, NCHD, chunk, 0)
    return k


def _tc_call(body, out_shapes):
    return pl.pallas_call(
        body,
        out_shape=[jax.ShapeDtypeStruct(s, d) for (s, d) in out_shapes],
    )


@functools.lru_cache(maxsize=None)
def _tc1_fn(N, NR):
    def body(x_ref, fcw_ref, fcb_ref, cw_ref, cas_ref, cad_ref,
             xfc_ref, h_ref, as_ref, ad_ref, b_ref):
        x = x_ref[...]
        xfc = jnp.dot(x, fcw_ref[...], preferred_element_type=F32) + fcb_ref[...]
        h = jnp.dot(xfc, cw_ref[...], preferred_element_type=F32)
        a_s = jnp.dot(h, cas_ref[...], preferred_element_type=F32)
        a_d = jnp.dot(h, cad_ref[...], preferred_element_type=F32)
        xfc_ref[...] = xfc
        h_ref[...] = h
        as_ref[...] = a_s
        ad_ref[...] = a_d
        mask = lax.broadcasted_iota(I32, (NR, 1), 0) < N
        b = (jnp.max(jnp.where(mask, a_s, -3e38))
             + jnp.max(jnp.where(mask, a_d, -3e38)))
        b_ref[...] = jnp.full((1, 128), b, F32)

    def run(x, fcw, fcb, cw, cas, cad):
        return _tc_call(body, [((NR, 128), F32), ((NR, 128), F32),
                               ((NR, 1), F32), ((NR, 1), F32),
                               ((1, 128), F32)])(x, fcw, fcb, cw, cas, cad)
    return run


@functools.lru_cache(maxsize=None)
def _tc2_fn(N, NR, ND, layer):
    def body(out0_ref, out1_ref, den0_ref, h_ref, xfc_ref,
             as_ref, ad_ref, b_ref, cb_ref, g_ref, bb_ref, wr_ref, wo_ref,
             xn_ref, r_ref, o_ref):
        bscal = b_ref[...][0, 0]
        h = h_ref[...]
        xfc = xfc_ref[...]
        es = as_ref[...] + ad_ref[...]
        es = jnp.where(es > 0, es, es * 0.2)
        ws = jnp.exp(es - bscal)
        outn = jnp.concatenate([out0_ref[...], out1_ref[...]], axis=1) + ws * h
        den = den0_ref[...] + ws
        xg = outn / (den + 1e-16) + cb_ref[...]
        mask = lax.broadcasted_iota(I32, (NR, 1), 0) < N
        maskf = mask.astype(F32)
        invn = (1.0 / N)

        def bn(z):
            mu = jnp.sum(z * maskf, axis=0, keepdims=True) * invn
            zc = z - mu
            var = jnp.sum(zc * zc * maskf, axis=0, keepdims=True) * invn
            return zc / jnp.sqrt(var + 1e-5) * g_ref[...] + bb_ref[...]

        if layer == 0:
            xn = xfc + bn(xg)
            xn = jnp.maximum(xn, 0.0)
        elif layer == 1:
            xn = bn(xfc) + xg
            xn = jnp.maximum(xn, 0.0)
        else:
            xn = bn(xfc) + xg
        xn = jnp.where(mask, xn, 0.0)
        xn_ref[...] = xn
        r_ref[...] = jnp.dot(xn, wr_ref[...], preferred_element_type=F32)
        o_ref[...] = jnp.dot(xn, wo_ref[...], preferred_element_type=F32)

    def run(out0, out1, den0, h, xfc, a_s, a_d, bvec, cb, g, bb, wr, wo):
        return _tc_call(body, [((NR, 128), F32), ((NR, 1), F32),
                               ((NR, 1), F32)])(
            out0, out1, den0, h, xfc, a_s, a_d, bvec, cb, g, bb, wr, wo)
    return run


@functools.lru_cache(maxsize=None)
def _tc3a_fn(NR):
    def body(a0_ref, a1_ref, o_ref, br_ref, pre_ref):
        pre_ref[...] = a0_ref[...] + a1_ref[...] + o_ref[...] + br_ref[...][0, 0]

    def run(a0, a1, o, br):
        return _tc_call(body, [((NR, 1), F32)])(a0, a1, o, br)[0]
    return run


@functools.lru_cache(maxsize=None)
def _rank_fn(N, NR):
    R = NR // 128

    def body(s_ref, s1_ref, rank_ref):
        ones_row = jnp.ones((1, 128), F32)
        ltriT = (lax.broadcasted_iota(I32, (128, 128), 0)
                 < lax.broadcasted_iota(I32, (128, 128), 1)).astype(F32)
        zmat = jnp.zeros((128, 128), F32)
        lane1 = lax.broadcasted_iota(I32, (1, 128), 1)
        sub1 = lax.broadcasted_iota(I32, (128, 1), 0)

        def irow(r, carry):
            si = s_ref[pl.ds(r, 1), :]
            ui = lax.bitcast_convert_type(si, jnp.uint32)
            ui = jnp.where(si < 0, ~ui, ui | jnp.uint32(0x80000000))
            ui = jnp.where(r * 128 + lane1 < N, ui, jnp.uint32(0))

            def jrow(r2, acc):
                sj = s1_ref[pl.ds(r2 * 128, 128), :]
                uj = lax.bitcast_convert_type(sj, jnp.uint32)
                uj = jnp.where(sj < 0, ~uj, uj | jnp.uint32(0x80000000))
                uj = jnp.where(r2 * 128 + sub1 < N, uj, jnp.uint32(0))
                gt = (uj > ui).astype(F32)
                eq = (uj == ui).astype(F32)
                low = jnp.where(r2 < r, eq, jnp.where(r2 == r, eq * ltriT, zmat))
                return acc + gt + low
            acc = lax.fori_loop(0, R, jrow, zmat)
            cnt = jnp.dot(ones_row, acc, preferred_element_type=F32)
            rank_ref[pl.ds(r, 1), :] = cnt.astype(I32)
            return carry
        lax.fori_loop(0, R, irow, 0)

    def run(s2, s1):
        return pl.pallas_call(
            body, out_shape=[jax.ShapeDtypeStruct((R, 128), I32)],
        )(s2, s1)[0]
    return run


@functools.lru_cache(maxsize=None)
def _tc3b_fn(N, NR, NP2, K):
    R = NR // 128
    RP = NP2 // 128

    def body(rank_ref, s_ref, gsel_ref, nix_ref, scat_ref):
        rank = rank_ref[...]
        s2 = s_ref[...]
        nidx = lax.broadcasted_iota(I32, (R, 128), 0) * 128 + \
            lax.broadcasted_iota(I32, (R, 128), 1)
        sel = (rank < K) & (nidx < N)
        gsel_ref[...] = jnp.where(sel, s2, 0.0)
        nix_ref[...] = jnp.where(sel, rank, -1)
        lanes = lax.broadcasted_iota(I32, (RP, 128), 1)
        nidx2 = lax.broadcasted_iota(I32, (RP, 128), 0) * 128 + lanes
        if RP > R:
            rankp = jnp.concatenate([rank, jnp.full((RP - R, 128), K, I32)])
        else:
            rankp = rank
        scat_ref[...] = jnp.where((rankp < K) & (nidx2 < N), rankp,
                                  K + (lanes & 127))

    def run(rank, s2):
        return _tc_call(body, [((R, 128), F32), ((R, 128), I32),
                               ((RP, 128), I32)])(rank, s2)
    return run


@functools.lru_cache(maxsize=None)
def _tc3final_fn(N, NR, K):
    R = NR // 128

    def body(rank_ref, s_ref, x_ref, out_ref, g_ref):
        rank = rank_ref[...]
        s2 = s_ref[...]
        nidx = lax.broadcasted_iota(I32, (R, 128), 0) * 128 + \
            lax.broadcasted_iota(I32, (R, 128), 1)
        sel = (rank < K) & (nidx < N)
        g_ref[...] = jnp.where(sel, s2, 0.0)

        def step(r, acc):
            grow = g_ref[pl.ds(r, 1), :]
            xblk = x_ref[pl.ds(r * 128, 128), :]
            return acc + jnp.dot(grow, xblk, preferred_element_type=F32)
        acc = lax.fori_loop(0, R, step, jnp.zeros((1, 128), F32))
        out_ref[...] = acc * (1.0 / K)

    def run(rank, s2, x):
        return pl.pallas_call(
            body,
            out_shape=[jax.ShapeDtypeStruct((1, 128), F32)],
            scratch_shapes=[pltpu.VMEM((R, 128), F32)],
        )(rank, s2, x)[0]
    return run


def _layer(x_pad, s3d, t3d, N, params, layer):
    (fcW, fcb, cW, cas, cad, cb, Wr, br, Wo, g, b) = params
    NR = _nr(N)
    ND = _nd2(N)
    K = (N + 1) // 2

    xfc, h, a_s, a_d, bvec = _tc1_fn(N, NR)(
        x_pad, fcW, fcb.reshape(1, 128), cW, cas.reshape(128, 1),
        cad.reshape(128, 1))

    as1 = a_s.reshape(NR)
    ad1 = a_d.reshape(NR)
    NCHT = EP // (16 * CH)
    sA = s3d.reshape(16, NCHT, CH)
    tA = t3d.reshape(16, NCHT, CH)
    outp, denp = _gat_edge_kernel(N, NR, ND)(
        sA, tA, as1, ad1, h[:, :64], h[:, 64:], bvec.reshape(128)[:16])
    def _rows(a, want):
        n0 = a.shape[0]
        if n0 >= want:
            return a[:want]
        pad_shape = (want - n0,) + a.shape[1:]
        return jnp.concatenate([a, jnp.zeros(pad_shape, F32)])

    out0 = _rows(outp[0], NR)
    out1 = _rows(outp[1], NR)
    den0 = _rows(denp[0], NR).reshape(NR, 1)

    xn, r1, o1 = _tc2_fn(N, NR, ND, layer)(
        out0, out1, den0, h, xfc, a_s, a_d, bvec,
        cb.reshape(1, 128), g.reshape(1, 128), b.reshape(1, 128), Wr, Wo)

    aggp = _pool_agg_kernel(N, NR, ND)(s3d, t3d, r1.reshape(NR))[0]
    a0 = _rows(aggp[0], NR).reshape(NR, 1)
    a1 = _rows(aggp[1], NR).reshape(NR, 1)
    pre = _tc3a_fn(NR)(a0, a1, o1, br.reshape(1, 1))
    score = jnp.tanh(pre)
    s2 = score.reshape(NR // 128, 128)
    rank = _rank_fn(N, NR)(s2, score)

    if layer == 2:
        return _tc3final_fn(N, NR, K)(rank, s2, xn), None, None, None

    NP2 = ((NR + NW * CH - 1) // (NW * CH)) * (NW * CH)
    NCHD = NP2 // (NW * CH)
    gsel, nix, scat = _tc3b_fn(N, NR, NP2, K)(rank, s2)
    x_for = jnp.concatenate(
        [xn, jnp.zeros((NP2 - NR, 128), F32)]) if NP2 > NR else xn
    g_for = (jnp.concatenate([gsel.reshape(NR), jnp.zeros((NP2 - NR,), F32)])
             if NP2 > NR else gsel.reshape(NR))
    y = _compact_kernel(NP2, NCHD, K)(
        x_for, g_for, scat.reshape(NW, NCHD, CH))[0]

    KR = _nr(K)
    x_next = y[:K]
    if KR > K:
        x_next = jnp.concatenate([x_next, jnp.zeros((KR - K, 128), F32)])

    s3d_n, t3d_n = _remap_kernel(N, NR, ND, K)(s3d, t3d, nix.reshape(NR))
    return None, x_next, s3d_n, t3d_n


def kernel(x, edge_index, edge_attr, batch,
           fc1_W, fc1_b, fc2_W, fc2_b, fc3_W, fc3_b,
           c1_W, c1_as, c1_ad, c1_b, c2_W, c2_as, c2_ad, c2_b,
           c3_W, c3_as, c3_ad, c3_b,
           p1_Wr, p1_br, p1_Wo, p2_Wr, p2_br, p2_Wo, p3_Wr, p3_br, p3_Wo,
           bn1_g, bn1_b, bn2_g, bn2_b, bn3_g, bn3_b):
    N = x.shape[0]
    NR = _nr(N)
    src = edge_index[0].astype(I32)
    dst = edge_index[1].astype(I32)
    E = src.shape[0]
    pad = EP - E
    ar = jnp.arange(pad, dtype=I32)
    s3d = jnp.concatenate([src, jnp.zeros((pad,), I32)]).reshape(NW, NCH, CH)
    t3d = jnp.concatenate([dst, N + (ar & 31)]).reshape(NW, NCH, CH)
    x_pad = jnp.concatenate([x, jnp.zeros((NR - N, 128), F32)]) if NR > N else x

    plist = [
        (fc1_W, fc1_b, c1_W, c1_as, c1_ad, c1_b, p1_Wr, p1_br, p1_Wo, bn1_g, bn1_b),
        (fc2_W, fc2_b, c2_W, c2_as, c2_ad, c2_b, p2_Wr, p2_br, p2_Wo, bn2_g, bn2_b),
        (fc3_W, fc3_b, c3_W, c3_as, c3_ad, c3_b, p3_Wr, p3_br, p3_Wo, bn3_g, bn3_b),
    ]
    for li in range(3):
        out, x_pad, s3d, t3d = _layer(x_pad, s3d, t3d, N, plist[li], li)
        if li == 2:
            return out
        N = (N + 1) // 2


# speedup vs baseline: 4.9705x; 1.0086x over previous
import functools
import math

import jax
import jax.numpy as jnp
from jax import lax
from jax.experimental import pallas as pl
from jax.experimental.pallas import tpu as pltpu
from jax.experimental.pallas import tpu_sc as plsc

F32 = jnp.float32
I32 = jnp.int32

NW = 32
CH = 128
NCH = 80
EPT = NCH * CH
EP = NW * EPT
E_REAL = 320000


def _nr(n):
    return ((n + 127) // 128) * 128


def _nd2(n):
    return ((n + 32 + 15) // 16) * 16


def _sc_mesh():
    return plsc.VectorSubcoreMesh(core_axis_name="c", subcore_axis_name="s")


@functools.lru_cache(maxsize=None)
def _gat_edge_kernel(N, NR, ND):
    NCHT = EP // (16 * CH)
    NDA = max(NR, ND)
    NB = ND // 128
    REM = ND - NB * 128
    NBT = (NB + 15) // 16

    @functools.partial(
        pl.kernel,
        mesh=_sc_mesh(),
        compiler_params=pltpu.CompilerParams(use_tc_tiling_on_sc=False, needs_layout_passes=False),
        out_type=[jax.ShapeDtypeStruct((2, ND, 64), F32),
                  jax.ShapeDtypeStruct((2, ND), F32)],
        scratch_types=[
            pltpu.VMEM((NCHT, CH), I32),
            pltpu.VMEM((NCHT, CH), I32),
            pltpu.VMEM((2 * CH,), F32),
            pltpu.VMEM((1024,), F32),
            pltpu.VMEM((NDA,), F32),
            pltpu.VMEM((NDA,), F32),
            pltpu.VMEM((16,), F32),
            pltpu.VMEM((CH, 64), F32),
            pltpu.VMEM((CH, 64), F32),
            pltpu.VMEM_SHARED((ND, 64), F32),
            pltpu.VMEM_SHARED((ND,), F32),
            pltpu.SemaphoreType.DMA,
            pltpu.SemaphoreType.DMA,
            pltpu.SemaphoreType.DMA,
            pltpu.SemaphoreType.DMA,
        ],
    )
    def k(s_hbm, t_hbm, as_hbm, ad_hbm, h0_hbm, h1_hbm, b_hbm, out_hbm,
          den_hbm, sv, tv, wv, zb, asv, adv, bv, rows0, rows1, out_sh, den_sh,
          sg0, sg1, ss0, ss1):
        cid = lax.axis_index("c")
        sid = lax.axis_index("s")

        pltpu.sync_copy(s_hbm.at[sid], sv)
        pltpu.sync_copy(t_hbm.at[sid], tv)
        pltpu.sync_copy(as_hbm, asv.at[pl.ds(0, NR)])
        pltpu.sync_copy(ad_hbm, adv.at[pl.ds(0, NR)])
        pltpu.sync_copy(b_hbm, bv)
        z16 = jnp.zeros((16,), F32)
        for i in range((NDA - NR) // 16):
            asv[pl.ds(NR + i * 16, 16)] = z16
            adv[pl.ds(NR + i * 16, 16)] = z16

        def zrow(i, c):
            rows0[i >> 2, pl.ds((i & 3) * 16, 16)] = z16
            return c
        lax.fori_loop(0, CH * 4, zrow, 0)
        def zw(i, c):
            zb[pl.ds(i * 16, 16)] = z16
            return c
        lax.fori_loop(0, 1024 // 16, zw, 0)
        for b in range(NBT):
            blk = b * 16 + sid
            @pl.when(blk < NB)
            def _():
                pltpu.sync_copy(rows0, out_sh.at[pl.ds(blk * 128, 128)])
        if REM:
            @pl.when(sid == 15)
            def _():
                pltpu.sync_copy(rows0.at[pl.ds(0, REM)],
                                out_sh.at[pl.ds(NB * 128, REM)])
        @pl.when(sid == 0)
        def _():
            for zi in range(ND // 1024):
                pltpu.sync_copy(zb, den_sh.at[pl.ds(zi * 1024, 1024)])
            zrem = ND - (ND // 1024) * 1024
            if zrem:
                pltpu.sync_copy(zb.at[pl.ds(0, zrem)],
                                den_sh.at[pl.ds((ND // 1024) * 1024, zrem)])
        plsc.subcore_barrier()

        bscal = bv[...][0]

        def mainloop(h_hbm, do_den):
            def compute_w(j, par):
                def wgrp(i, c2):
                    svv = sv[j, pl.ds(i * 16, 16)]
                    tvv = tv[j, pl.ds(i * 16, 16)]
                    a = plsc.load_gather(asv, [svv])
                    b2 = plsc.load_gather(adv, [tvv])
                    e = a + b2
                    e = jnp.where(e > 0, e, e * 0.2)
                    w = jnp.exp(e - bscal)
                    wv[pl.ds(par * CH + i * 16, 16)] = w
                    return c2
                lax.fori_loop(0, CH // 16, wgrp, 0)

            def scale(par, rows):
                def s4(e2, c3):
                    base = par * CH + e2 * 4
                    for q in range(4):
                        wbc = plsc.load_gather(
                            wv, [jnp.full((16,), base + q, I32)])
                        for kk in range(4):
                            rows[e2 * 4 + q, pl.ds(kk * 16, 16)] = (
                                rows[e2 * 4 + q, pl.ds(kk * 16, 16)] * wbc)
                    return c3
                lax.fori_loop(0, CH // 4, s4, 0)

            def pair(i, c):
                j0 = i * 2
                j1 = j0 + 1
                g0 = pltpu.async_copy(h_hbm.at[sv.at[j0]], rows0, sg0)
                g1 = pltpu.async_copy(h_hbm.at[sv.at[j1]], rows1, sg1)
                compute_w(j0, 0)
                compute_w(j1, 1)
                g0.wait()
                scale(0, rows0)
                s0 = pltpu.async_copy(rows0, out_sh.at[tv.at[j0]], ss0, add=True)
                if do_den:
                    pltpu.sync_copy(wv.at[pl.ds(0, CH)],
                                    den_sh.at[tv.at[j0]], add=True)
                g1.wait()
                scale(1, rows1)
                s0.wait()
                s1 = pltpu.async_copy(rows1, out_sh.at[tv.at[j1]], ss1, add=True)
                if do_den:
                    pltpu.sync_copy(wv.at[pl.ds(CH, CH)],
                                    den_sh.at[tv.at[j1]], add=True)
                s1.wait()
                return c
            lax.fori_loop(0, NCHT // 2, pair, 0)

        @pl.when(cid == 0)
        def _():
            mainloop(h0_hbm, True)
        @pl.when(cid == 1)
        def _():
            mainloop(h1_hbm, False)

        plsc.subcore_barrier()
        for b in range(NBT):
            blk = b * 16 + sid
            @pl.when(blk < NB)
            def _():
                pltpu.sync_copy(out_sh.at[pl.ds(blk * 128, 128)],
                                out_hbm.at[cid, pl.ds(blk * 128, 128)])
        if REM:
            @pl.when(sid == 15)
            def _():
                pltpu.sync_copy(out_sh.at[pl.ds(NB * 128, REM)],
                                out_hbm.at[cid, pl.ds(NB * 128, REM)])
        @pl.when(sid == 0)
        def _():
            pltpu.sync_copy(den_sh, den_hbm.at[cid])
    return k


@functools.lru_cache(maxsize=None)
def _pool_agg_kernel(N, NR, ND):
    @functools.partial(
        pl.kernel,
        mesh=_sc_mesh(),
        compiler_params=pltpu.CompilerParams(use_tc_tiling_on_sc=False, needs_layout_passes=False),
        out_type=[jax.ShapeDtypeStruct((2, ND), F32)],
        scratch_types=[
            pltpu.VMEM((NCH, CH), I32),
            pltpu.VMEM((NCH, CH), I32),
            pltpu.VMEM((NCH * CH,), F32),
            pltpu.VMEM((ND,), F32),
            pltpu.VMEM_SHARED((ND,), F32),
        ],
    )
    def k(s_hbm, t_hbm, r_hbm, agg_hbm, sv, tv, wv, rv, agg_sh):
        cid = lax.axis_index("c")
        sid = lax.axis_index("s")
        wid = sid * 2 + cid
        pltpu.sync_copy(s_hbm.at[wid], sv)
        pltpu.sync_copy(t_hbm.at[wid], tv)
        pltpu.sync_copy(r_hbm, rv.at[pl.ds(0, NR)])
        z16 = jnp.zeros((16,), F32)
        def zw(i, c):
            wv[pl.ds(i * 16, 16)] = z16
            return c
        lax.fori_loop(0, NCH * CH // 16, zw, 0)
        @pl.when(sid == 0)
        def _():
            pltpu.sync_copy(wv.at[pl.ds(0, ND)], agg_sh)
        plsc.subcore_barrier()

        def chunk(j, c):
            def wgrp(i, c2):
                svv = sv[j, pl.ds(i * 16, 16)]
                w = plsc.load_gather(rv, [svv])
                wv[pl.ds(j * CH + i * 16, 16)] = w
                return c2
            lax.fori_loop(0, CH // 16, wgrp, 0)
            pltpu.sync_copy(wv.at[pl.ds(j * CH, CH)], agg_sh.at[tv.at[j]], add=True)
            return c
        lax.fori_loop(0, NCH, chunk, 0)

        plsc.subcore_barrier()
        @pl.when(sid == 0)
        def _():
            pltpu.sync_copy(agg_sh, agg_hbm.at[cid])
    return k


@functools.lru_cache(maxsize=None)
def _remap_kernel(N, NR, ND, KN):
    NDA = max(NR, ND)

    @functools.partial(
        pl.kernel,
        mesh=_sc_mesh(),
        compiler_params=pltpu.CompilerParams(use_tc_tiling_on_sc=False, needs_layout_passes=False),
        out_type=[jax.ShapeDtypeStruct((NW, NCH, CH), I32),
                  jax.ShapeDtypeStruct((NW, NCH, CH), I32)],
        scratch_types=[
            pltpu.VMEM((NCH, CH), I32),
            pltpu.VMEM((NCH, CH), I32),
            pltpu.VMEM((NDA,), I32),
        ],
    )
    def k(s_hbm, t_hbm, nix_hbm, so_hbm, to_hbm, sv, tv, nixv):
        cid = lax.axis_index("c")
        sid = lax.axis_index("s")
        wid = sid * 2 + cid
        pltpu.sync_copy(s_hbm.at[wid], sv)
        pltpu.sync_copy(t_hbm.at[wid], tv)
        pltpu.sync_copy(nix_hbm, nixv.at[pl.ds(0, NR)])
        m16 = jnp.full((16,), -1, I32)
        for i in range((NDA - NR) // 16):
            nixv[pl.ds(NR + i * 16, 16)] = m16
        lanes = lax.iota(I32, 16)

        def chunk(j, c):
            def grp(i, c2):
                svv = sv[j, pl.ds(i * 16, 16)]
                tvv = tv[j, pl.ds(i * 16, 16)]
                ns = plsc.load_gather(nixv, [svv])
                nt = plsc.load_gather(nixv, [tvv])
                ok = (ns >= 0) & (nt >= 0)
                dummy = KN + ((i * 16 + lanes) & 31)
                sv[j, pl.ds(i * 16, 16)] = jnp.where(ok, ns, 0)
                tv[j, pl.ds(i * 16, 16)] = jnp.where(ok, nt, dummy)
                return c2
            lax.fori_loop(0, CH // 16, grp, 0)
            return c
        lax.fori_loop(0, NCH, chunk, 0)
        pltpu.sync_copy(sv, so_hbm.at[wid])
        pltpu.sync_copy(tv, to_hbm.at[wid])
    return k


@functools.lru_cache(maxsize=None)
def _compact_kernel(NP2, NCHD, KN):
    @functools.partial(
        pl.kernel,
        mesh=_sc_mesh(),
        compiler_params=pltpu.CompilerParams(use_tc_tiling_on_sc=False, needs_layout_passes=False),
        out_type=[jax.ShapeDtypeStruct((KN + 128, 128), F32)],
        scratch_types=[
            pltpu.VMEM((NCHD, CH), I32),
            pltpu.VMEM((NCHD * CH,), F32),
            pltpu.VMEM((CH, 128), F32),
        ],
    )
    def k(x_hbm, g_hbm, scat_hbm, y_hbm, scv, gv, rows):
        cid = lax.axis_index("c")
        sid = lax.axis_index("s")
        wid = sid * 2 + cid
        base = wid * NCHD * CH
        pltpu.sync_copy(scat_hbm.at[wid], scv)
        pltpu.sync_copy(g_hbm.at[pl.ds(base, NCHD * CH)], gv)

        def chunk(j, c):
            pltpu.sync_copy(x_hbm.at[pl.ds(base + j * CH, CH)], rows)
            def scale(e2, c3):
                wbc = plsc.load_gather(gv, [jnp.full((16,), j * CH + e2, I32)])
                def f8(kk, c4):
                    rows[e2, pl.ds(kk * 16, 16)] = rows[e2, pl.ds(kk * 16, 16)] * wbc
                    return c4
                lax.fori_loop(0, 8, f8, 0)
                return c3
            lax.fori_loop(0, CH, scale, 0)
            pltpu.sync_copy(rows, y_hbm.at[scv.at[j]])
            return c
        lax.fori_loop(0, NCHD, chunk, 0)
    return k


def _tc_call(body, out_shapes):
    return pl.pallas_call(
        body,
        out_shape=[jax.ShapeDtypeStruct(s, d) for (s, d) in out_shapes],
    )


@functools.lru_cache(maxsize=None)
def _tc1_fn(N, NR):
    def body(x_ref, fcw_ref, fcb_ref, cw_ref, cas_ref, cad_ref,
             xfc_ref, h_ref, as_ref, ad_ref, b_ref):
        x = x_ref[...]
        xfc = jnp.dot(x, fcw_ref[...], preferred_element_type=F32) + fcb_ref[...]
        h = jnp.dot(xfc, cw_ref[...], preferred_element_type=F32)
        a_s = jnp.dot(h, cas_ref[...], preferred_element_type=F32)
        a_d = jnp.dot(h, cad_ref[...], preferred_element_type=F32)
        xfc_ref[...] = xfc
        h_ref[...] = h
        as_ref[...] = a_s
        ad_ref[...] = a_d
        mask = lax.broadcasted_iota(I32, (NR, 1), 0) < N
        b = (jnp.max(jnp.where(mask, a_s, -3e38))
             + jnp.max(jnp.where(mask, a_d, -3e38)))
        b_ref[...] = jnp.full((1, 128), b, F32)

    def run(x, fcw, fcb, cw, cas, cad):
        return _tc_call(body, [((NR, 128), F32), ((NR, 128), F32),
                               ((NR, 1), F32), ((NR, 1), F32),
                               ((1, 128), F32)])(x, fcw, fcb, cw, cas, cad)
    return run


@functools.lru_cache(maxsize=None)
def _tc2_fn(N, NR, ND, layer):
    def body(out0_ref, out1_ref, den0_ref, h_ref, xfc_ref,
             as_ref, ad_ref, b_ref, cb_ref, g_ref, bb_ref, wr_ref, wo_ref,
             xn_ref, r_ref, o_ref):
        bscal = b_ref[...][0, 0]
        h = h_ref[...]
        xfc = xfc_ref[...]
        es = as_ref[...] + ad_ref[...]
        es = jnp.where(es > 0, es, es * 0.2)
        ws = jnp.exp(es - bscal)
        outn = jnp.concatenate([out0_ref[...], out1_ref[...]], axis=1) + ws * h
        den = den0_ref[...] + ws
        xg = outn / (den + 1e-16) + cb_ref[...]
        mask = lax.broadcasted_iota(I32, (NR, 1), 0) < N
        maskf = mask.astype(F32)
        invn = (1.0 / N)

        def bn(z):
            mu = jnp.sum(z * maskf, axis=0, keepdims=True) * invn
            zc = z - mu
            var = jnp.sum(zc * zc * maskf, axis=0, keepdims=True) * invn
            return zc / jnp.sqrt(var + 1e-5) * g_ref[...] + bb_ref[...]

        if layer == 0:
            xn = xfc + bn(xg)
            xn = jnp.maximum(xn, 0.0)
        elif layer == 1:
            xn = bn(xfc) + xg
            xn = jnp.maximum(xn, 0.0)
        else:
            xn = bn(xfc) + xg
        xn = jnp.where(mask, xn, 0.0)
        xn_ref[...] = xn
        r_ref[...] = jnp.dot(xn, wr_ref[...], preferred_element_type=F32)
        o_ref[...] = jnp.dot(xn, wo_ref[...], preferred_element_type=F32)

    def run(out0, out1, den0, h, xfc, a_s, a_d, bvec, cb, g, bb, wr, wo):
        return _tc_call(body, [((NR, 128), F32), ((NR, 1), F32),
                               ((NR, 1), F32)])(
            out0, out1, den0, h, xfc, a_s, a_d, bvec, cb, g, bb, wr, wo)
    return run


@functools.lru_cache(maxsize=None)
def _tc3a_fn(NR):
    def body(a0_ref, a1_ref, o_ref, br_ref, pre_ref):
        pre_ref[...] = a0_ref[...] + a1_ref[...] + o_ref[...] + br_ref[...][0, 0]

    def run(a0, a1, o, br):
        return _tc_call(body, [((NR, 1), F32)])(a0, a1, o, br)[0]
    return run


@functools.lru_cache(maxsize=None)
def _rank_fn(N, NR):
    R = NR // 128

    def body(s_ref, s1_ref, rank_ref):
        ones_row = jnp.ones((1, 128), F32)
        ltriT = (lax.broadcasted_iota(I32, (128, 128), 0)
                 < lax.broadcasted_iota(I32, (128, 128), 1)).astype(F32)
        zmat = jnp.zeros((128, 128), F32)
        lane1 = lax.broadcasted_iota(I32, (1, 128), 1)
        sub1 = lax.broadcasted_iota(I32, (128, 1), 0)

        def irow(r, carry):
            si = s_ref[pl.ds(r, 1), :]
            ui = lax.bitcast_convert_type(si, jnp.uint32)
            ui = jnp.where(si < 0, ~ui, ui | jnp.uint32(0x80000000))
            ui = jnp.where(r * 128 + lane1 < N, ui, jnp.uint32(0))

            def jrow(r2, acc):
                sj = s1_ref[pl.ds(r2 * 128, 128), :]
                uj = lax.bitcast_convert_type(sj, jnp.uint32)
                uj = jnp.where(sj < 0, ~uj, uj | jnp.uint32(0x80000000))
                uj = jnp.where(r2 * 128 + sub1 < N, uj, jnp.uint32(0))
                gt = (uj > ui).astype(F32)
                eq = (uj == ui).astype(F32)
                low = jnp.where(r2 < r, eq, jnp.where(r2 == r, eq * ltriT, zmat))
                return acc + gt + low
            acc = lax.fori_loop(0, R, jrow, zmat)
            cnt = jnp.dot(ones_row, acc, preferred_element_type=F32)
            rank_ref[pl.ds(r, 1), :] = cnt.astype(I32)
            return carry
        lax.fori_loop(0, R, irow, 0)

    def run(s2, s1):
        return pl.pallas_call(
            body, out_shape=[jax.ShapeDtypeStruct((R, 128), I32)],
        )(s2, s1)[0]
    return run


@functools.lru_cache(maxsize=None)
def _tc3b_fn(N, NR, NP2, K):
    R = NR // 128
    RP = NP2 // 128

    def body(rank_ref, s_ref, gsel_ref, nix_ref, scat_ref):
        rank = rank_ref[...]
        s2 = s_ref[...]
        nidx = lax.broadcasted_iota(I32, (R, 128), 0) * 128 + \
            lax.broadcasted_iota(I32, (R, 128), 1)
        sel = (rank < K) & (nidx < N)
        gsel_ref[...] = jnp.where(sel, s2, 0.0)
        nix_ref[...] = jnp.where(sel, rank, -1)
        lanes = lax.broadcasted_iota(I32, (RP, 128), 1)
        nidx2 = lax.broadcasted_iota(I32, (RP, 128), 0) * 128 + lanes
        if RP > R:
            rankp = jnp.concatenate([rank, jnp.full((RP - R, 128), K, I32)])
        else:
            rankp = rank
        scat_ref[...] = jnp.where((rankp < K) & (nidx2 < N), rankp,
                                  K + (lanes & 127))

    def run(rank, s2):
        return _tc_call(body, [((R, 128), F32), ((R, 128), I32),
                               ((RP, 128), I32)])(rank, s2)
    return run


@functools.lru_cache(maxsize=None)
def _tc3final_fn(N, NR, K):
    R = NR // 128

    def body(rank_ref, s_ref, x_ref, out_ref, g_ref):
        rank = rank_ref[...]
        s2 = s_ref[...]
        nidx = lax.broadcasted_iota(I32, (R, 128), 0) * 128 + \
            lax.broadcasted_iota(I32, (R, 128), 1)
        sel = (rank < K) & (nidx < N)
        g_ref[...] = jnp.where(sel, s2, 0.0)

        def step(r, acc):
            grow = g_ref[pl.ds(r, 1), :]
            xblk = x_ref[pl.ds(r * 128, 128), :]
            return acc + jnp.dot(grow, xblk, preferred_element_type=F32)
        acc = lax.fori_loop(0, R, step, jnp.zeros((1, 128), F32))
        out_ref[...] = acc * (1.0 / K)

    def run(rank, s2, x):
        return pl.pallas_call(
            body,
            out_shape=[jax.ShapeDtypeStruct((1, 128), F32)],
            scratch_shapes=[pltpu.VMEM((R, 128), F32)],
        )(rank, s2, x)[0]
    return run


def _layer(x_pad, s3d, t3d, N, params, layer):
    (fcW, fcb, cW, cas, cad, cb, Wr, br, Wo, g, b) = params
    NR = _nr(N)
    ND = _nd2(N)
    K = (N + 1) // 2

    xfc, h, a_s, a_d, bvec = _tc1_fn(N, NR)(
        x_pad, fcW, fcb.reshape(1, 128), cW, cas.reshape(128, 1),
        cad.reshape(128, 1))

    as1 = a_s.reshape(NR)
    ad1 = a_d.reshape(NR)
    NCHT = EP // (16 * CH)
    sA = s3d.reshape(16, NCHT, CH)
    tA = t3d.reshape(16, NCHT, CH)
    outp, denp = _gat_edge_kernel(N, NR, ND)(
        sA, tA, as1, ad1, h[:, :64], h[:, 64:], bvec.reshape(128)[:16])
    def _rows(a, want):
        n0 = a.shape[0]
        if n0 >= want:
            return a[:want]
        pad_shape = (want - n0,) + a.shape[1:]
        return jnp.concatenate([a, jnp.zeros(pad_shape, F32)])

    out0 = _rows(outp[0], NR)
    out1 = _rows(outp[1], NR)
    den0 = _rows(denp[0], NR).reshape(NR, 1)

    xn, r1, o1 = _tc2_fn(N, NR, ND, layer)(
        out0, out1, den0, h, xfc, a_s, a_d, bvec,
        cb.reshape(1, 128), g.reshape(1, 128), b.reshape(1, 128), Wr, Wo)

    aggp = _pool_agg_kernel(N, NR, ND)(s3d, t3d, r1.reshape(NR))[0]
    a0 = _rows(aggp[0], NR).reshape(NR, 1)
    a1 = _rows(aggp[1], NR).reshape(NR, 1)
    pre = _tc3a_fn(NR)(a0, a1, o1, br.reshape(1, 1))
    score = jnp.tanh(pre)
    s2 = score.reshape(NR // 128, 128)
    rank = _rank_fn(N, NR)(s2, score)

    if layer == 2:
        return _tc3final_fn(N, NR, K)(rank, s2, xn), None, None, None

    NP2 = ((NR + NW * CH - 1) // (NW * CH)) * (NW * CH)
    NCHD = NP2 // (NW * CH)
    gsel, nix, scat = _tc3b_fn(N, NR, NP2, K)(rank, s2)
    x_for = jnp.concatenate(
        [xn, jnp.zeros((NP2 - NR, 128), F32)]) if NP2 > NR else xn
    g_for = (jnp.concatenate([gsel.reshape(NR), jnp.zeros((NP2 - NR,), F32)])
             if NP2 > NR else gsel.reshape(NR))
    y = _compact_kernel(NP2, NCHD, K)(
        x_for, g_for, scat.reshape(NW, NCHD, CH))[0]

    KR = _nr(K)
    x_next = y[:K]
    if KR > K:
        x_next = jnp.concatenate([x_next, jnp.zeros((KR - K, 128), F32)])

    s3d_n, t3d_n = _remap_kernel(N, NR, ND, K)(s3d, t3d, nix.reshape(NR))
    return None, x_next, s3d_n, t3d_n


def kernel(x, edge_index, edge_attr, batch,
           fc1_W, fc1_b, fc2_W, fc2_b, fc3_W, fc3_b,
           c1_W, c1_as, c1_ad, c1_b, c2_W, c2_as, c2_ad, c2_b,
           c3_W, c3_as, c3_ad, c3_b,
           p1_Wr, p1_br, p1_Wo, p2_Wr, p2_br, p2_Wo, p3_Wr, p3_br, p3_Wo,
           bn1_g, bn1_b, bn2_g, bn2_b, bn3_g, bn3_b):
    N = x.shape[0]
    NR = _nr(N)
    src = edge_index[0].astype(I32)
    dst = edge_index[1].astype(I32)
    E = src.shape[0]
    pad = EP - E
    ar = jnp.arange(pad, dtype=I32)
    s3d = jnp.concatenate([src, jnp.zeros((pad,), I32)]).reshape(NW, NCH, CH)
    t3d = jnp.concatenate([dst, N + (ar & 31)]).reshape(NW, NCH, CH)
    x_pad = jnp.concatenate([x, jnp.zeros((NR - N, 128), F32)]) if NR > N else x

    plist = [
        (fc1_W, fc1_b, c1_W, c1_as, c1_ad, c1_b, p1_Wr, p1_br, p1_Wo, bn1_g, bn1_b),
        (fc2_W, fc2_b, c2_W, c2_as, c2_ad, c2_b, p2_Wr, p2_br, p2_Wo, bn2_g, bn2_b),
        (fc3_W, fc3_b, c3_W, c3_as, c3_ad, c3_b, p3_Wr, p3_br, p3_Wo, bn3_g, bn3_b),
    ]
    for li in range(3):
        out, x_pad, s3d, t3d = _layer(x_pad, s3d, t3d, N, plist[li], li)
        if li == 2:
            return out
        N = (N + 1) // 2
